# baseline (speedup 1.0000x reference)
import functools

import jax
import jax.numpy as jnp
from jax import lax
from jax.experimental import pallas as pl
from jax.experimental.pallas import tpu as pltpu
from jax.experimental.pallas import tpu_sc as plsc

N = 8192
D = 768
KNN = 10
ALPHA = 0.99
NU = 1.0 - ALPHA
NCLS = 1000
NITER = 10

NW = 32
RP = N // NW
NE = N * KNN
LANE = 16
CB = 128
NB = 8
CPAD = NB * CB

CS_SCAN = 4096
SCAN_BUF = 8320
CAP = 93184
SCH = 2048

_mesh = plsc.VectorSubcoreMesh(core_axis_name="c", subcore_axis_name="s")


def _yk_body(x_ref, w_ref, y_ref, f_ref):
    x = x_ref[...]
    nrm = jnp.sqrt(jnp.sum(x * x, axis=1, keepdims=True))
    f = x / (nrm + 1e-12)
    f_ref[...] = f
    logits = lax.dot_general(f, w_ref[...], (((1,), (1,)), ((), ())),
                             preferred_element_type=jnp.float32)
    m = jnp.max(logits, axis=1, keepdims=True)
    e = jnp.exp(logits - m)
    y_ref[...] = e / jnp.sum(e, axis=1, keepdims=True)


def _topk_body(f_ref, ft_ref, w_ref, i_ref, s_ref):
    i = pl.program_id(0)
    bi = f_ref.shape[0]
    s_ref[...] = lax.dot_general(f_ref[...], ft_ref[...],
                                 (((1,), (0,)), ((), ())),
                                 preferred_element_type=jnp.float32)
    row_g = lax.broadcasted_iota(jnp.int32, (bi, N), 0) + i * bi
    col = lax.broadcasted_iota(jnp.int32, (bi, N), 1)
    s_ref[...] = jnp.where(col == row_g, -10.0, s_ref[...])
    for k in range(KNN):
        s = s_ref[...]
        m = jnp.max(s, axis=1, keepdims=True)
        am = jnp.min(jnp.where(s == m, col, N), axis=1, keepdims=True)
        w_ref[:, k] = jnp.maximum(m[:, 0], 0.0) ** 3
        i_ref[:, k] = am[:, 0]
        s_ref[...] = jnp.where(col == am, -1e30, s)
    for k in range(KNN, 16):
        w_ref[:, k] = jnp.zeros((bi,), jnp.float32)
        i_ref[:, k] = jnp.zeros((bi,), jnp.int32)


def _dinv_body(p_ref, o_ref):
    deg = jnp.sum(p_ref[...], axis=0, keepdims=True)
    o_ref[...] = 1.0 / jnp.sqrt(deg + 1e-12)


def _wid():
    return lax.axis_index("s") * 2 + lax.axis_index("c")


@functools.partial(
    pl.kernel, mesh=_mesh,
    compiler_params=pltpu.CompilerParams(needs_layout_passes=False),
    out_type=jax.ShapeDtypeStruct((NW, N), jnp.float32),
    scratch_types=[
        pltpu.VMEM((NE // NW,), jnp.int32),
        pltpu.VMEM((NE // NW,), jnp.float32),
        pltpu.VMEM((NE // NW,), jnp.int32),
        pltpu.VMEM((N,), jnp.float32),
    ])
def _deg_kernel(idx_hbm, w_hbm, rows_hbm, degp_hbm, idxs, ws, rs, deg):
    wid = _wid()
    sl = NE // NW
    base = wid * sl
    pltpu.sync_copy(idx_hbm.at[pl.ds(base, sl)], idxs)
    pltpu.sync_copy(w_hbm.at[pl.ds(base, sl)], ws)
    pltpu.sync_copy(rows_hbm.at[pl.ds(base, sl)], rs)

    def zero_body(i, _):
        deg[pl.ds(i * LANE, LANE)] = jnp.zeros((LANE,), jnp.float32)
        return 0

    lax.fori_loop(0, N // LANE, zero_body, 0)

    def body(v, _):
        iv = idxs[pl.ds(v * LANE, LANE)]
        wv = ws[pl.ds(v * LANE, LANE)]
        rv = rs[pl.ds(v * LANE, LANE)]
        plsc.addupdate_scatter(deg, [iv], wv)
        plsc.addupdate_scatter(deg, [rv], wv)
        return 0

    lax.fori_loop(0, sl // LANE, body, 0)
    pltpu.sync_copy(deg, degp_hbm.at[wid])


@functools.partial(
    pl.kernel, mesh=_mesh,
    compiler_params=pltpu.CompilerParams(needs_layout_passes=False),
    out_type=(jax.ShapeDtypeStruct((NW, CAP), jnp.int32),
              jax.ShapeDtypeStruct((NW, CAP), jnp.int32),
              jax.ShapeDtypeStruct((NW, CAP), jnp.float32),
              jax.ShapeDtypeStruct((NW * 128,), jnp.int32)),
    scratch_types=[
        pltpu.VMEM((CS_SCAN,), jnp.int32),
        pltpu.VMEM((CS_SCAN,), jnp.float32),
        pltpu.VMEM((CS_SCAN,), jnp.int32),
        pltpu.VMEM((N,), jnp.float32),
        pltpu.VMEM((SCAN_BUF,), jnp.int32),
        pltpu.VMEM((SCAN_BUF,), jnp.int32),
        pltpu.VMEM((SCAN_BUF,), jnp.float32),
        pltpu.VMEM((LANE,), jnp.int32),
    ])
def _build_kernel(idx_hbm, w_hbm, rows_hbm, dinv_hbm, eld_hbm, els_hbm,
                  elc_hbm, cnt_hbm, idxs, ws, rws, dinv, bd, bs, bc, cvec):
    wid = _wid()
    lo = wid * RP
    pltpu.sync_copy(dinv_hbm, dinv)
    lanes = lax.iota(jnp.int32, LANE)
    nchunk = NE // CS_SCAN

    def chunk_body(ch, carry):
        cnt, off = carry
        pltpu.sync_copy(idx_hbm.at[pl.ds(ch * CS_SCAN, CS_SCAN)], idxs)
        pltpu.sync_copy(w_hbm.at[pl.ds(ch * CS_SCAN, CS_SCAN)], ws)
        pltpu.sync_copy(rows_hbm.at[pl.ds(ch * CS_SCAN, CS_SCAN)], rws)

        def vec_body(v, cnt):
            iv = idxs[pl.ds(v * LANE, LANE)]
            wv = ws[pl.ds(v * LANE, LANE)]
            rv = rws[pl.ds(v * LANE, LANE)]
            di = plsc.load_gather(dinv, [iv])
            dr = plsc.load_gather(dinv, [rv])
            coef = (ALPHA * wv) * di * dr
            m1 = (iv >= lo) & (iv < lo + RP)
            plsc.store_compressed(bd.at[pl.ds(cnt, LANE)], iv - lo, mask=m1)
            plsc.store_compressed(bs.at[pl.ds(cnt, LANE)], rv, mask=m1)
            plsc.store_compressed(bc.at[pl.ds(cnt, LANE)], coef, mask=m1)
            cnt = cnt + jnp.sum(m1.astype(jnp.int32), axis=0)
            m2 = (rv >= lo) & (rv < lo + RP)
            plsc.store_compressed(bd.at[pl.ds(cnt, LANE)], rv - lo, mask=m2)
            plsc.store_compressed(bs.at[pl.ds(cnt, LANE)], iv, mask=m2)
            plsc.store_compressed(bc.at[pl.ds(cnt, LANE)], coef, mask=m2)
            cnt = cnt + jnp.sum(m2.astype(jnp.int32), axis=0)
            return cnt

        cnt = lax.fori_loop(0, CS_SCAN // LANE, vec_body, cnt)
        offa = pl.multiple_of(off, 128)
        pltpu.sync_copy(bd, eld_hbm.at[wid].at[pl.ds(offa, SCAN_BUF)])
        pltpu.sync_copy(bs, els_hbm.at[wid].at[pl.ds(offa, SCAN_BUF)])
        pltpu.sync_copy(bc, elc_hbm.at[wid].at[pl.ds(offa, SCAN_BUF)])
        c128 = (cnt // 128) * 128
        rem = cnt - c128
        for j in range(8):
            rd = bd[pl.ds(c128 + j * LANE, LANE)]
            rs = bs[pl.ds(c128 + j * LANE, LANE)]
            rc = bc[pl.ds(c128 + j * LANE, LANE)]
            bd[pl.ds(j * LANE, LANE)] = rd
            bs[pl.ds(j * LANE, LANE)] = rs
            bc[pl.ds(j * LANE, LANE)] = rc
        return (rem, off + c128)

    cnt, off = lax.fori_loop(0, nchunk, chunk_body,
                             (jnp.int32(0), jnp.int32(0)))
    cv16 = (cnt + LANE - 1) // LANE

    def zb(i, _):
        @pl.when(i >= cv16)
        def _z():
            bd[pl.ds(i * LANE, LANE)] = jnp.zeros((LANE,), jnp.int32)
            bs[pl.ds(i * LANE, LANE)] = jnp.zeros((LANE,), jnp.int32)
            bc[pl.ds(i * LANE, LANE)] = jnp.zeros((LANE,), jnp.float32)
        return 0

    lax.fori_loop(0, SCAN_BUF // LANE, zb, 0)
    c16 = (cnt // LANE) * LANE
    tl = cnt - c16
    mtail = lanes < tl
    zd = jnp.where(mtail, bd[pl.ds(c16, LANE)], 0)
    zs = jnp.where(mtail, bs[pl.ds(c16, LANE)], 0)
    zc = jnp.where(mtail, bc[pl.ds(c16, LANE)], 0.0)
    bd[pl.ds(c16, LANE)] = zd
    bs[pl.ds(c16, LANE)] = zs
    bc[pl.ds(c16, LANE)] = zc
    offa = pl.multiple_of(off, 128)
    pltpu.sync_copy(bd, eld_hbm.at[wid].at[pl.ds(offa, SCAN_BUF)])
    pltpu.sync_copy(bs, els_hbm.at[wid].at[pl.ds(offa, SCAN_BUF)])
    pltpu.sync_copy(bc, elc_hbm.at[wid].at[pl.ds(offa, SCAN_BUF)])
    n_total = off + cnt
    cvec[pl.ds(0, LANE)] = jnp.full((LANE,), n_total, jnp.int32)
    pltpu.sync_copy(cvec, cnt_hbm.at[pl.ds(wid * 128, LANE)])


@functools.partial(
    pl.kernel, mesh=_mesh,
    compiler_params=pltpu.CompilerParams(needs_layout_passes=False),
    out_type=jax.ShapeDtypeStruct((NB, N * CB), jnp.float32),
    scratch_types=[
        pltpu.VMEM((SCH,), jnp.int32),
        pltpu.VMEM((SCH,), jnp.int32),
        pltpu.VMEM((SCH,), jnp.float32),
        pltpu.VMEM((NW * 128,), jnp.int32),
        pltpu.VMEM((128, CB), jnp.float32),
        pltpu.VMEM((128, CB), jnp.float32),
        pltpu.VMEM((RP * CB,), jnp.float32),
        pltpu.VMEM((RP * CB,), jnp.float32),
        pltpu.SemaphoreType.DMA,
        pltpu.SemaphoreType.DMA,
    ])
def _prop_kernel(z_hbm, y_hbm, eld_hbm, els_hbm, elc_hbm, cnt_hbm, zo_hbm,
                 eld, els, elc, cbuf, g0, g1, acc, ybuf, sem0, sem1):
    wid = _wid()
    lo = wid * RP
    lanes = lax.iota(jnp.int32, LANE)
    pltpu.sync_copy(cnt_hbm, cbuf)
    nt = jnp.max(cbuf[pl.ds(wid * 128, LANE)], axis=0)
    nsc = (nt + SCH - 1) // SCH

    def process(gb, gsem, gbatch):
        base = gbatch * 128
        coefs = [elc[pl.ds(base + g * LANE, LANE)] for g in range(8)]
        dbases = [eld[pl.ds(base + g * LANE, LANE)] * CB for g in range(8)]
        rows = [lanes + g * LANE for g in range(8)]

        def cbody(c, _):
            colv = jnp.full((LANE,), 0, jnp.int32) + c
            for g in range(8):
                val = plsc.load_gather(gb, [rows[g], colv])
                plsc.addupdate_scatter(acc, [dbases[g] + c], val * coefs[g])
            return 0

        lax.fori_loop(0, CB, cbody, 0)

    def block_body(b, _):
        def zacc(i, _):
            acc[pl.ds(i * LANE, LANE)] = jnp.zeros((LANE,), jnp.float32)
            return 0

        lax.fori_loop(0, RP * CB // LANE, zacc, 0)

        def sc_body(s, _):
            soff = s * SCH
            pltpu.sync_copy(eld_hbm.at[wid].at[pl.ds(soff, SCH)], eld)
            pltpu.sync_copy(els_hbm.at[wid].at[pl.ds(soff, SCH)], els)
            pltpu.sync_copy(elc_hbm.at[wid].at[pl.ds(soff, SCH)], elc)
            nb_s = jnp.minimum(nt - soff, SCH)
            nbat = (nb_s + 127) // 128

            @pl.when(s == nsc - 1)
            def _fix():
                def fixv(v, _):
                    pos = soff + v * LANE + lanes
                    e = els[pl.ds(v * LANE, LANE)]
                    els[pl.ds(v * LANE, LANE)] = jnp.where(
                        pos < nt, e, pos & (N - 1))
                    return 0

                lax.fori_loop(0, SCH // LANE, fixv, 0)

            pltpu.make_async_copy(
                z_hbm.at[b].at[els.at[pl.ds(0, 128)]], g0, sem0).start()

            def batch_body(g, _):
                @pl.when(g % 2 == 0)
                def _even():
                    pltpu.make_async_copy(
                        z_hbm.at[b].at[els.at[pl.ds(g * 128, 128)]],
                        g0, sem0).wait()

                    @pl.when(g + 1 < nbat)
                    def _fire():
                        pltpu.make_async_copy(
                            z_hbm.at[b].at[els.at[pl.ds((g + 1) * 128, 128)]],
                            g1, sem1).start()

                    process(g0, sem0, g)

                @pl.when(g % 2 == 1)
                def _odd():
                    pltpu.make_async_copy(
                        z_hbm.at[b].at[els.at[pl.ds(g * 128, 128)]],
                        g1, sem1).wait()

                    @pl.when(g + 1 < nbat)
                    def _fire():
                        pltpu.make_async_copy(
                            z_hbm.at[b].at[els.at[pl.ds((g + 1) * 128, 128)]],
                            g0, sem0).start()

                    process(g1, sem1, g)

                return 0

            lax.fori_loop(0, nbat, batch_body, 0)
            return 0

        lax.fori_loop(0, nsc, sc_body, 0)

        pltpu.sync_copy(y_hbm.at[b].at[pl.ds(lo * CB, RP * CB)], ybuf)

        def yadd(i, _):
            av = acc[pl.ds(i * LANE, LANE)]
            yv = ybuf[pl.ds(i * LANE, LANE)]
            acc[pl.ds(i * LANE, LANE)] = av + NU * yv
            return 0

        lax.fori_loop(0, RP * CB // LANE, yadd, 0)
        pltpu.sync_copy(acc, zo_hbm.at[b].at[pl.ds(lo * CB, RP * CB)])
        return 0

    lax.fori_loop(0, NB, block_body, 0)


def kernel(features, classification_weight):
    nby = 16
    biy = N // nby
    Y, F = pl.pallas_call(
        _yk_body,
        grid=(nby,),
        in_specs=[pl.BlockSpec((biy, D), lambda i: (i, 0)),
                  pl.BlockSpec((NCLS, D), lambda i: (0, 0))],
        out_specs=[pl.BlockSpec((biy, NCLS), lambda i: (i, 0)),
                   pl.BlockSpec((biy, D), lambda i: (i, 0))],
        out_shape=[jax.ShapeDtypeStruct((N, NCLS), jnp.float32),
                   jax.ShapeDtypeStruct((N, D), jnp.float32)],
    )(features, classification_weight)

    nbk = 32
    bik = N // nbk
    Wk, Ik = pl.pallas_call(
        _topk_body,
        grid=(nbk,),
        in_specs=[pl.BlockSpec((bik, D), lambda i: (i, 0)),
                  pl.BlockSpec((D, N), lambda i: (0, 0))],
        out_specs=[pl.BlockSpec((bik, 16), lambda i: (i, 0)),
                   pl.BlockSpec((bik, 16), lambda i: (i, 0))],
        out_shape=[jax.ShapeDtypeStruct((N, 16), jnp.float32),
                   jax.ShapeDtypeStruct((N, 16), jnp.int32)],
        scratch_shapes=[pltpu.VMEM((bik, N), jnp.float32)],
    )(F, F.T)

    idxf = Ik[:, :KNN].reshape(-1)
    wf = Wk[:, :KNN].reshape(-1)
    rowsf = jnp.repeat(jnp.arange(N, dtype=jnp.int32), KNN)

    degp = _deg_kernel(idxf, wf, rowsf)
    dinv2 = pl.pallas_call(
        _dinv_body,
        out_shape=jax.ShapeDtypeStruct((1, N), jnp.float32),
    )(degp)
    dinv = dinv2.reshape(N)

    eld, els, elc, cnts = _build_kernel(idxf, wf, rowsf, dinv)

    ypad = jnp.pad(Y, ((0, 0), (0, CPAD - NCLS)))
    y3 = ypad.reshape(N, NB, CB).transpose(1, 0, 2)
    y2 = y3.reshape(NB, N * CB)

    z3 = y3
    for _ in range(NITER):
        zo = _prop_kernel(z3, y2, eld, els, elc, cnts)
        z3 = zo.reshape(NB, N, CB)

    zfin = z3.transpose(1, 0, 2).reshape(N, CPAD)[:, :NCLS]
    return zfin


# speedup vs baseline: 4.1410x; 4.1410x over previous
import functools

import jax
import jax.numpy as jnp
from jax import lax
from jax.experimental import pallas as pl
from jax.experimental.pallas import tpu as pltpu
from jax.experimental.pallas import tpu_sc as plsc

N = 8192
D = 768
KNN = 10
ALPHA = 0.99
NU = 1.0 - ALPHA
NCLS = 1000
NITER = 10

NW = 32
RP = N // NW
NE = N * KNN
LANE = 16
CB = 128
NB = 8
CPAD = NB * CB

CS_SCAN = 4096
SCAN_BUF = 8320
CAP = 93184
SCH = 2048

_mesh = plsc.VectorSubcoreMesh(core_axis_name="c", subcore_axis_name="s")


def _yk_body(x_ref, w_ref, y_ref, f_ref):
    x = x_ref[...]
    nrm = jnp.sqrt(jnp.sum(x * x, axis=1, keepdims=True))
    f = x / (nrm + 1e-12)
    f_ref[...] = f
    logits = lax.dot_general(f, w_ref[...], (((1,), (1,)), ((), ())),
                             preferred_element_type=jnp.float32)
    m = jnp.max(logits, axis=1, keepdims=True)
    e = jnp.exp(logits - m)
    y_ref[...] = e / jnp.sum(e, axis=1, keepdims=True)


def _topk_body(f_ref, ft_ref, w_ref, i_ref, s_ref):
    i = pl.program_id(0)
    bi = f_ref.shape[0]
    s_ref[...] = lax.dot_general(f_ref[...], ft_ref[...],
                                 (((1,), (0,)), ((), ())),
                                 preferred_element_type=jnp.float32)
    row_g = lax.broadcasted_iota(jnp.int32, (bi, N), 0) + i * bi
    col = lax.broadcasted_iota(jnp.int32, (bi, N), 1)
    s_ref[...] = jnp.where(col == row_g, -10.0, s_ref[...])
    for k in range(KNN):
        s = s_ref[...]
        m = jnp.max(s, axis=1, keepdims=True)
        am = jnp.min(jnp.where(s == m, col, N), axis=1, keepdims=True)
        w_ref[:, k] = jnp.maximum(m[:, 0], 0.0) ** 3
        i_ref[:, k] = am[:, 0]
        s_ref[...] = jnp.where(col == am, -1e30, s)
    for k in range(KNN, 16):
        w_ref[:, k] = jnp.zeros((bi,), jnp.float32)
        i_ref[:, k] = jnp.zeros((bi,), jnp.int32)


def _dinv_body(p_ref, o_ref):
    deg = jnp.sum(p_ref[...], axis=0, keepdims=True)
    o_ref[...] = 1.0 / jnp.sqrt(deg + 1e-12)


def _wid():
    return lax.axis_index("s") * 2 + lax.axis_index("c")


@functools.partial(
    pl.kernel, mesh=_mesh,
    compiler_params=pltpu.CompilerParams(needs_layout_passes=False),
    out_type=jax.ShapeDtypeStruct((NW, N), jnp.float32),
    scratch_types=[
        pltpu.VMEM((NE // NW,), jnp.int32),
        pltpu.VMEM((NE // NW,), jnp.float32),
        pltpu.VMEM((NE // NW,), jnp.int32),
        pltpu.VMEM((N,), jnp.float32),
    ])
def _deg_kernel(idx_hbm, w_hbm, rows_hbm, degp_hbm, idxs, ws, rs, deg):
    wid = _wid()
    sl = NE // NW
    base = wid * sl
    pltpu.sync_copy(idx_hbm.at[pl.ds(base, sl)], idxs)
    pltpu.sync_copy(w_hbm.at[pl.ds(base, sl)], ws)
    pltpu.sync_copy(rows_hbm.at[pl.ds(base, sl)], rs)

    def zero_body(i, _):
        deg[pl.ds(i * LANE, LANE)] = jnp.zeros((LANE,), jnp.float32)
        return 0

    lax.fori_loop(0, N // LANE, zero_body, 0)

    def body(v, _):
        iv = idxs[pl.ds(v * LANE, LANE)]
        wv = ws[pl.ds(v * LANE, LANE)]
        rv = rs[pl.ds(v * LANE, LANE)]
        plsc.addupdate_scatter(deg, [iv], wv)
        plsc.addupdate_scatter(deg, [rv], wv)
        return 0

    lax.fori_loop(0, sl // LANE, body, 0)
    pltpu.sync_copy(deg, degp_hbm.at[wid])


@functools.partial(
    pl.kernel, mesh=_mesh,
    compiler_params=pltpu.CompilerParams(needs_layout_passes=False),
    out_type=(jax.ShapeDtypeStruct((NW, CAP), jnp.int32),
              jax.ShapeDtypeStruct((NW, CAP), jnp.int32),
              jax.ShapeDtypeStruct((NW, CAP), jnp.float32),
              jax.ShapeDtypeStruct((NW * 128,), jnp.int32)),
    scratch_types=[
        pltpu.VMEM((CS_SCAN,), jnp.int32),
        pltpu.VMEM((CS_SCAN,), jnp.float32),
        pltpu.VMEM((CS_SCAN,), jnp.int32),
        pltpu.VMEM((N,), jnp.float32),
        pltpu.VMEM((SCAN_BUF,), jnp.int32),
        pltpu.VMEM((SCAN_BUF,), jnp.int32),
        pltpu.VMEM((SCAN_BUF,), jnp.float32),
        pltpu.VMEM((LANE,), jnp.int32),
    ])
def _build_kernel(idx_hbm, w_hbm, rows_hbm, dinv_hbm, eld_hbm, els_hbm,
                  elc_hbm, cnt_hbm, idxs, ws, rws, dinv, bd, bs, bc, cvec):
    wid = _wid()
    lo = wid * RP
    pltpu.sync_copy(dinv_hbm, dinv)
    lanes = lax.iota(jnp.int32, LANE)
    nchunk = NE // CS_SCAN

    def chunk_body(ch, carry):
        cnt, off = carry
        pltpu.sync_copy(idx_hbm.at[pl.ds(ch * CS_SCAN, CS_SCAN)], idxs)
        pltpu.sync_copy(w_hbm.at[pl.ds(ch * CS_SCAN, CS_SCAN)], ws)
        pltpu.sync_copy(rows_hbm.at[pl.ds(ch * CS_SCAN, CS_SCAN)], rws)

        def vec_body(v, cnt):
            iv = idxs[pl.ds(v * LANE, LANE)]
            wv = ws[pl.ds(v * LANE, LANE)]
            rv = rws[pl.ds(v * LANE, LANE)]
            di = plsc.load_gather(dinv, [iv])
            dr = plsc.load_gather(dinv, [rv])
            coef = (ALPHA * wv) * di * dr
            m1 = (iv >= lo) & (iv < lo + RP)
            plsc.store_compressed(bd.at[pl.ds(cnt, LANE)], iv - lo, mask=m1)
            plsc.store_compressed(bs.at[pl.ds(cnt, LANE)], rv, mask=m1)
            plsc.store_compressed(bc.at[pl.ds(cnt, LANE)], coef, mask=m1)
            cnt = cnt + jnp.sum(m1.astype(jnp.int32), axis=0)
            m2 = (rv >= lo) & (rv < lo + RP)
            plsc.store_compressed(bd.at[pl.ds(cnt, LANE)], rv - lo, mask=m2)
            plsc.store_compressed(bs.at[pl.ds(cnt, LANE)], iv, mask=m2)
            plsc.store_compressed(bc.at[pl.ds(cnt, LANE)], coef, mask=m2)
            cnt = cnt + jnp.sum(m2.astype(jnp.int32), axis=0)
            return cnt

        cnt = lax.fori_loop(0, CS_SCAN // LANE, vec_body, cnt)
        offa = pl.multiple_of(off, 128)
        pltpu.sync_copy(bd, eld_hbm.at[wid].at[pl.ds(offa, SCAN_BUF)])
        pltpu.sync_copy(bs, els_hbm.at[wid].at[pl.ds(offa, SCAN_BUF)])
        pltpu.sync_copy(bc, elc_hbm.at[wid].at[pl.ds(offa, SCAN_BUF)])
        c128 = (cnt // 128) * 128
        rem = cnt - c128
        for j in range(8):
            rd = bd[pl.ds(c128 + j * LANE, LANE)]
            rs = bs[pl.ds(c128 + j * LANE, LANE)]
            rc = bc[pl.ds(c128 + j * LANE, LANE)]
            bd[pl.ds(j * LANE, LANE)] = rd
            bs[pl.ds(j * LANE, LANE)] = rs
            bc[pl.ds(j * LANE, LANE)] = rc
        return (rem, off + c128)

    cnt, off = lax.fori_loop(0, nchunk, chunk_body,
                             (jnp.int32(0), jnp.int32(0)))
    cv16 = (cnt + LANE - 1) // LANE

    def zb(i, _):
        @pl.when(i >= cv16)
        def _z():
            bd[pl.ds(i * LANE, LANE)] = jnp.zeros((LANE,), jnp.int32)
            bs[pl.ds(i * LANE, LANE)] = jnp.zeros((LANE,), jnp.int32)
            bc[pl.ds(i * LANE, LANE)] = jnp.zeros((LANE,), jnp.float32)
        return 0

    lax.fori_loop(0, SCAN_BUF // LANE, zb, 0)
    c16 = (cnt // LANE) * LANE
    tl = cnt - c16
    mtail = lanes < tl
    zd = jnp.where(mtail, bd[pl.ds(c16, LANE)], 0)
    zs = jnp.where(mtail, bs[pl.ds(c16, LANE)], 0)
    zc = jnp.where(mtail, bc[pl.ds(c16, LANE)], 0.0)
    bd[pl.ds(c16, LANE)] = zd
    bs[pl.ds(c16, LANE)] = zs
    bc[pl.ds(c16, LANE)] = zc
    offa = pl.multiple_of(off, 128)
    pltpu.sync_copy(bd, eld_hbm.at[wid].at[pl.ds(offa, SCAN_BUF)])
    pltpu.sync_copy(bs, els_hbm.at[wid].at[pl.ds(offa, SCAN_BUF)])
    pltpu.sync_copy(bc, elc_hbm.at[wid].at[pl.ds(offa, SCAN_BUF)])
    n_total = off + cnt
    cvec[pl.ds(0, LANE)] = jnp.full((LANE,), n_total, jnp.int32)
    pltpu.sync_copy(cvec, cnt_hbm.at[pl.ds(wid * 128, LANE)])


@functools.partial(
    pl.kernel, mesh=_mesh,
    compiler_params=pltpu.CompilerParams(needs_layout_passes=False),
    out_type=jax.ShapeDtypeStruct((NB, N * CB), jnp.float32),
    scratch_types=[
        pltpu.VMEM((SCH,), jnp.int32),
        pltpu.VMEM((SCH,), jnp.int32),
        pltpu.VMEM((SCH,), jnp.float32),
        pltpu.VMEM((NW * 128,), jnp.int32),
        pltpu.VMEM((128, CB), jnp.float32),
        pltpu.VMEM((128, CB), jnp.float32),
        pltpu.VMEM((RP * CB,), jnp.float32),
        pltpu.VMEM((RP * CB,), jnp.float32),
        pltpu.SemaphoreType.DMA,
        pltpu.SemaphoreType.DMA,
    ])
def _prop_kernel(z_hbm, y_hbm, eld_hbm, els_hbm, elc_hbm, cnt_hbm, zo_hbm,
                 eld, els, elc, cbuf, g0, g1, acc, ybuf, sem0, sem1):
    wid = _wid()
    lo = wid * RP
    lanes = lax.iota(jnp.int32, LANE)
    pltpu.sync_copy(cnt_hbm, cbuf)
    nt = jnp.max(cbuf[pl.ds(wid * 128, LANE)], axis=0)
    nsc = (nt + SCH - 1) // SCH

    def process(gb, gsem, gbatch):
        base = gbatch * 128
        coefs = [elc[pl.ds(base + g * LANE, LANE)] for g in range(8)]
        dbases = [eld[pl.ds(base + g * LANE, LANE)] * CB for g in range(8)]
        rows = [lanes + g * LANE for g in range(8)]

        def cbody(c, _):
            cv = (lanes + c) & (CB - 1)
            for g in range(8):
                val = plsc.load_gather(gb, [rows[g], cv])
                plsc.addupdate_scatter(acc, [dbases[g] + cv], val * coefs[g])
            return 0

        lax.fori_loop(0, CB, cbody, 0)

    def block_body(b, _):
        def zacc(i, _):
            acc[pl.ds(i * LANE, LANE)] = jnp.zeros((LANE,), jnp.float32)
            return 0

        lax.fori_loop(0, RP * CB // LANE, zacc, 0)

        def sc_body(s, _):
            soff = s * SCH
            pltpu.sync_copy(eld_hbm.at[wid].at[pl.ds(soff, SCH)], eld)
            pltpu.sync_copy(els_hbm.at[wid].at[pl.ds(soff, SCH)], els)
            pltpu.sync_copy(elc_hbm.at[wid].at[pl.ds(soff, SCH)], elc)
            nb_s = jnp.minimum(nt - soff, SCH)
            nbat = (nb_s + 127) // 128

            @pl.when(s == nsc - 1)
            def _fix():
                def fixv(v, _):
                    pos = soff + v * LANE + lanes
                    e = els[pl.ds(v * LANE, LANE)]
                    els[pl.ds(v * LANE, LANE)] = jnp.where(
                        pos < nt, e, pos & (N - 1))
                    return 0

                lax.fori_loop(0, SCH // LANE, fixv, 0)

            pltpu.make_async_copy(
                z_hbm.at[b].at[els.at[pl.ds(0, 128)]], g0, sem0).start()

            def batch_body(g, _):
                @pl.when(g % 2 == 0)
                def _even():
                    pltpu.make_async_copy(
                        z_hbm.at[b].at[els.at[pl.ds(g * 128, 128)]],
                        g0, sem0).wait()

                    @pl.when(g + 1 < nbat)
                    def _fire():
                        pltpu.make_async_copy(
                            z_hbm.at[b].at[els.at[pl.ds((g + 1) * 128, 128)]],
                            g1, sem1).start()

                    process(g0, sem0, g)

                @pl.when(g % 2 == 1)
                def _odd():
                    pltpu.make_async_copy(
                        z_hbm.at[b].at[els.at[pl.ds(g * 128, 128)]],
                        g1, sem1).wait()

                    @pl.when(g + 1 < nbat)
                    def _fire():
                        pltpu.make_async_copy(
                            z_hbm.at[b].at[els.at[pl.ds((g + 1) * 128, 128)]],
                            g0, sem0).start()

                    process(g1, sem1, g)

                return 0

            lax.fori_loop(0, nbat, batch_body, 0)
            return 0

        lax.fori_loop(0, nsc, sc_body, 0)

        pltpu.sync_copy(y_hbm.at[b].at[pl.ds(lo * CB, RP * CB)], ybuf)

        def yadd(i, _):
            av = acc[pl.ds(i * LANE, LANE)]
            yv = ybuf[pl.ds(i * LANE, LANE)]
            acc[pl.ds(i * LANE, LANE)] = av + NU * yv
            return 0

        lax.fori_loop(0, RP * CB // LANE, yadd, 0)
        pltpu.sync_copy(acc, zo_hbm.at[b].at[pl.ds(lo * CB, RP * CB)])
        return 0

    lax.fori_loop(0, NB, block_body, 0)


def kernel(features, classification_weight):
    nby = 16
    biy = N // nby
    Y, F = pl.pallas_call(
        _yk_body,
        grid=(nby,),
        in_specs=[pl.BlockSpec((biy, D), lambda i: (i, 0)),
                  pl.BlockSpec((NCLS, D), lambda i: (0, 0))],
        out_specs=[pl.BlockSpec((biy, NCLS), lambda i: (i, 0)),
                   pl.BlockSpec((biy, D), lambda i: (i, 0))],
        out_shape=[jax.ShapeDtypeStruct((N, NCLS), jnp.float32),
                   jax.ShapeDtypeStruct((N, D), jnp.float32)],
    )(features, classification_weight)

    nbk = 32
    bik = N // nbk
    Wk, Ik = pl.pallas_call(
        _topk_body,
        grid=(nbk,),
        in_specs=[pl.BlockSpec((bik, D), lambda i: (i, 0)),
                  pl.BlockSpec((D, N), lambda i: (0, 0))],
        out_specs=[pl.BlockSpec((bik, 16), lambda i: (i, 0)),
                   pl.BlockSpec((bik, 16), lambda i: (i, 0))],
        out_shape=[jax.ShapeDtypeStruct((N, 16), jnp.float32),
                   jax.ShapeDtypeStruct((N, 16), jnp.int32)],
        scratch_shapes=[pltpu.VMEM((bik, N), jnp.float32)],
    )(F, F.T)

    idxf = Ik[:, :KNN].reshape(-1)
    wf = Wk[:, :KNN].reshape(-1)
    rowsf = jnp.repeat(jnp.arange(N, dtype=jnp.int32), KNN)

    degp = _deg_kernel(idxf, wf, rowsf)
    dinv2 = pl.pallas_call(
        _dinv_body,
        out_shape=jax.ShapeDtypeStruct((1, N), jnp.float32),
    )(degp)
    dinv = dinv2.reshape(N)

    eld, els, elc, cnts = _build_kernel(idxf, wf, rowsf, dinv)

    ypad = jnp.pad(Y, ((0, 0), (0, CPAD - NCLS)))
    y3 = ypad.reshape(N, NB, CB).transpose(1, 0, 2)
    y2 = y3.reshape(NB, N * CB)

    z3 = y3
    for _ in range(NITER):
        zo = _prop_kernel(z3, y2, eld, els, elc, cnts)
        z3 = zo.reshape(NB, N, CB)

    zfin = z3.transpose(1, 0, 2).reshape(N, CPAD)[:, :NCLS]
    return zfin


# speedup vs baseline: 4.2594x; 1.0286x over previous
import functools

import jax
import jax.numpy as jnp
from jax import lax
from jax.experimental import pallas as pl
from jax.experimental.pallas import tpu as pltpu
from jax.experimental.pallas import tpu_sc as plsc

N = 8192
D = 768
KNN = 10
ALPHA = 0.99
NU = 1.0 - ALPHA
NCLS = 1000
NITER = 10

NW = 32
RP = N // NW
NE = N * KNN
LANE = 16
CB = 256
NB = 4
CPAD = NB * CB

CS_SCAN = 4096
SCAN_BUF = 8320
CAP = 93184
SCH = 2048
GB = 64

_mesh = plsc.VectorSubcoreMesh(core_axis_name="c", subcore_axis_name="s")


def _yk_body(x_ref, w_ref, y_ref, f_ref):
    x = x_ref[...]
    nrm = jnp.sqrt(jnp.sum(x * x, axis=1, keepdims=True))
    f = x / (nrm + 1e-12)
    f_ref[...] = f
    logits = lax.dot_general(f, w_ref[...], (((1,), (1,)), ((), ())),
                             preferred_element_type=jnp.float32)
    m = jnp.max(logits, axis=1, keepdims=True)
    e = jnp.exp(logits - m)
    y_ref[...] = e / jnp.sum(e, axis=1, keepdims=True)


def _topk_body(f_ref, ft_ref, w_ref, i_ref, s_ref):
    i = pl.program_id(0)
    bi = f_ref.shape[0]
    s_ref[...] = lax.dot_general(f_ref[...], ft_ref[...],
                                 (((1,), (0,)), ((), ())),
                                 preferred_element_type=jnp.float32)
    row_g = lax.broadcasted_iota(jnp.int32, (bi, N), 0) + i * bi
    col = lax.broadcasted_iota(jnp.int32, (bi, N), 1)
    s_ref[...] = jnp.where(col == row_g, -10.0, s_ref[...])
    for k in range(KNN):
        s = s_ref[...]
        m = jnp.max(s, axis=1, keepdims=True)
        am = jnp.min(jnp.where(s == m, col, N), axis=1, keepdims=True)
        w_ref[:, k] = jnp.maximum(m[:, 0], 0.0) ** 3
        i_ref[:, k] = am[:, 0]
        s_ref[...] = jnp.where(col == am, -1e30, s)
    for k in range(KNN, 16):
        w_ref[:, k] = jnp.zeros((bi,), jnp.float32)
        i_ref[:, k] = jnp.zeros((bi,), jnp.int32)


def _dinv_body(p_ref, o_ref):
    deg = jnp.sum(p_ref[...], axis=0, keepdims=True)
    o_ref[...] = 1.0 / jnp.sqrt(deg + 1e-12)


def _wid():
    return lax.axis_index("s") * 2 + lax.axis_index("c")


@functools.partial(
    pl.kernel, mesh=_mesh,
    compiler_params=pltpu.CompilerParams(needs_layout_passes=False),
    out_type=jax.ShapeDtypeStruct((NW, N), jnp.float32),
    scratch_types=[
        pltpu.VMEM((NE // NW,), jnp.int32),
        pltpu.VMEM((NE // NW,), jnp.float32),
        pltpu.VMEM((NE // NW,), jnp.int32),
        pltpu.VMEM((N,), jnp.float32),
    ])
def _deg_kernel(idx_hbm, w_hbm, rows_hbm, degp_hbm, idxs, ws, rs, deg):
    wid = _wid()
    sl = NE // NW
    base = wid * sl
    pltpu.sync_copy(idx_hbm.at[pl.ds(base, sl)], idxs)
    pltpu.sync_copy(w_hbm.at[pl.ds(base, sl)], ws)
    pltpu.sync_copy(rows_hbm.at[pl.ds(base, sl)], rs)

    def zero_body(i, _):
        deg[pl.ds(i * LANE, LANE)] = jnp.zeros((LANE,), jnp.float32)
        return 0

    lax.fori_loop(0, N // LANE, zero_body, 0)

    def body(v, _):
        iv = idxs[pl.ds(v * LANE, LANE)]
        wv = ws[pl.ds(v * LANE, LANE)]
        rv = rs[pl.ds(v * LANE, LANE)]
        plsc.addupdate_scatter(deg, [iv], wv)
        plsc.addupdate_scatter(deg, [rv], wv)
        return 0

    lax.fori_loop(0, sl // LANE, body, 0)
    pltpu.sync_copy(deg, degp_hbm.at[wid])


@functools.partial(
    pl.kernel, mesh=_mesh,
    compiler_params=pltpu.CompilerParams(needs_layout_passes=False),
    out_type=(jax.ShapeDtypeStruct((NW, CAP), jnp.int32),
              jax.ShapeDtypeStruct((NW, CAP), jnp.int32),
              jax.ShapeDtypeStruct((NW, CAP), jnp.float32),
              jax.ShapeDtypeStruct((NW * 128,), jnp.int32)),
    scratch_types=[
        pltpu.VMEM((CS_SCAN,), jnp.int32),
        pltpu.VMEM((CS_SCAN,), jnp.float32),
        pltpu.VMEM((CS_SCAN,), jnp.int32),
        pltpu.VMEM((N,), jnp.float32),
        pltpu.VMEM((SCAN_BUF,), jnp.int32),
        pltpu.VMEM((SCAN_BUF,), jnp.int32),
        pltpu.VMEM((SCAN_BUF,), jnp.float32),
        pltpu.VMEM((LANE,), jnp.int32),
    ])
def _build_kernel(idx_hbm, w_hbm, rows_hbm, dinv_hbm, eld_hbm, els_hbm,
                  elc_hbm, cnt_hbm, idxs, ws, rws, dinv, bd, bs, bc, cvec):
    wid = _wid()
    lo = wid * RP
    pltpu.sync_copy(dinv_hbm, dinv)
    lanes = lax.iota(jnp.int32, LANE)
    nchunk = NE // CS_SCAN

    def chunk_body(ch, carry):
        cnt, off = carry
        pltpu.sync_copy(idx_hbm.at[pl.ds(ch * CS_SCAN, CS_SCAN)], idxs)
        pltpu.sync_copy(w_hbm.at[pl.ds(ch * CS_SCAN, CS_SCAN)], ws)
        pltpu.sync_copy(rows_hbm.at[pl.ds(ch * CS_SCAN, CS_SCAN)], rws)

        def vec_body(v, cnt):
            iv = idxs[pl.ds(v * LANE, LANE)]
            wv = ws[pl.ds(v * LANE, LANE)]
            rv = rws[pl.ds(v * LANE, LANE)]
            di = plsc.load_gather(dinv, [iv])
            dr = plsc.load_gather(dinv, [rv])
            coef = (ALPHA * wv) * di * dr
            m1 = (iv >= lo) & (iv < lo + RP)
            plsc.store_compressed(bd.at[pl.ds(cnt, LANE)], iv - lo, mask=m1)
            plsc.store_compressed(bs.at[pl.ds(cnt, LANE)], rv, mask=m1)
            plsc.store_compressed(bc.at[pl.ds(cnt, LANE)], coef, mask=m1)
            cnt = cnt + jnp.sum(m1.astype(jnp.int32), axis=0)
            m2 = (rv >= lo) & (rv < lo + RP)
            plsc.store_compressed(bd.at[pl.ds(cnt, LANE)], rv - lo, mask=m2)
            plsc.store_compressed(bs.at[pl.ds(cnt, LANE)], iv, mask=m2)
            plsc.store_compressed(bc.at[pl.ds(cnt, LANE)], coef, mask=m2)
            cnt = cnt + jnp.sum(m2.astype(jnp.int32), axis=0)
            return cnt

        cnt = lax.fori_loop(0, CS_SCAN // LANE, vec_body, cnt)
        offa = pl.multiple_of(off, 128)
        pltpu.sync_copy(bd, eld_hbm.at[wid].at[pl.ds(offa, SCAN_BUF)])
        pltpu.sync_copy(bs, els_hbm.at[wid].at[pl.ds(offa, SCAN_BUF)])
        pltpu.sync_copy(bc, elc_hbm.at[wid].at[pl.ds(offa, SCAN_BUF)])
        c128 = (cnt // 128) * 128
        rem = cnt - c128
        for j in range(8):
            rd = bd[pl.ds(c128 + j * LANE, LANE)]
            rs = bs[pl.ds(c128 + j * LANE, LANE)]
            rc = bc[pl.ds(c128 + j * LANE, LANE)]
            bd[pl.ds(j * LANE, LANE)] = rd
            bs[pl.ds(j * LANE, LANE)] = rs
            bc[pl.ds(j * LANE, LANE)] = rc
        return (rem, off + c128)

    cnt, off = lax.fori_loop(0, nchunk, chunk_body,
                             (jnp.int32(0), jnp.int32(0)))
    cv16 = (cnt + LANE - 1) // LANE

    def zb(i, _):
        @pl.when(i >= cv16)
        def _z():
            bd[pl.ds(i * LANE, LANE)] = jnp.zeros((LANE,), jnp.int32)
            bs[pl.ds(i * LANE, LANE)] = jnp.zeros((LANE,), jnp.int32)
            bc[pl.ds(i * LANE, LANE)] = jnp.zeros((LANE,), jnp.float32)
        return 0

    lax.fori_loop(0, SCAN_BUF // LANE, zb, 0)
    c16 = (cnt // LANE) * LANE
    tl = cnt - c16
    mtail = lanes < tl
    zd = jnp.where(mtail, bd[pl.ds(c16, LANE)], 0)
    zs = jnp.where(mtail, bs[pl.ds(c16, LANE)], 0)
    zc = jnp.where(mtail, bc[pl.ds(c16, LANE)], 0.0)
    bd[pl.ds(c16, LANE)] = zd
    bs[pl.ds(c16, LANE)] = zs
    bc[pl.ds(c16, LANE)] = zc
    offa = pl.multiple_of(off, 128)
    pltpu.sync_copy(bd, eld_hbm.at[wid].at[pl.ds(offa, SCAN_BUF)])
    pltpu.sync_copy(bs, els_hbm.at[wid].at[pl.ds(offa, SCAN_BUF)])
    pltpu.sync_copy(bc, elc_hbm.at[wid].at[pl.ds(offa, SCAN_BUF)])
    n_total = off + cnt
    cvec[pl.ds(0, LANE)] = jnp.full((LANE,), n_total, jnp.int32)
    pltpu.sync_copy(cvec, cnt_hbm.at[pl.ds(wid * 128, LANE)])


@functools.partial(
    pl.kernel, mesh=_mesh,
    compiler_params=pltpu.CompilerParams(needs_layout_passes=False),
    out_type=jax.ShapeDtypeStruct((NB, N * CB), jnp.float32),
    scratch_types=[
        pltpu.VMEM((SCH,), jnp.int32),
        pltpu.VMEM((SCH,), jnp.int32),
        pltpu.VMEM((SCH,), jnp.float32),
        pltpu.VMEM((NW * 128,), jnp.int32),
        pltpu.VMEM((GB, CB), jnp.float32),
        pltpu.VMEM((GB, CB), jnp.float32),
        pltpu.VMEM((GB, CB), jnp.float32),
        pltpu.VMEM((RP * CB,), jnp.float32),
        pltpu.SemaphoreType.DMA,
        pltpu.SemaphoreType.DMA,
        pltpu.SemaphoreType.DMA,
    ])
def _prop_kernel(z_hbm, y_hbm, eld_hbm, els_hbm, elc_hbm, cnt_hbm, zo_hbm,
                 eld, els, elc, cbuf, g0, g1, g2, acc, sem0, sem1, sem2):
    wid = _wid()
    lo = wid * RP
    lanes = lax.iota(jnp.int32, LANE)
    pltpu.sync_copy(cnt_hbm, cbuf)
    nt = jnp.max(cbuf[pl.ds(wid * 128, LANE)], axis=0)
    nsc = (nt + SCH - 1) // SCH
    ngrp = GB // LANE

    def process(gb, gbatch):
        base = gbatch * GB
        coefs = [elc[pl.ds(base + g * LANE, LANE)] for g in range(ngrp)]
        dbases = [eld[pl.ds(base + g * LANE, LANE)] * CB for g in range(ngrp)]
        rows = [lanes + g * LANE for g in range(ngrp)]

        def cbody(c, _):
            cv = (lanes + c) & (CB - 1)
            for g in range(ngrp):
                val = plsc.load_gather(gb, [rows[g], cv])
                plsc.addupdate_scatter(acc, [dbases[g] + cv], val * coefs[g])
            return 0

        lax.fori_loop(0, CB, cbody, 0)

    def block_body(b, _):
        pltpu.sync_copy(y_hbm.at[b].at[pl.ds(lo * CB, RP * CB)], acc)

        def yscale(i, _):
            acc[pl.ds(i * LANE, LANE)] = acc[pl.ds(i * LANE, LANE)] * NU
            return 0

        lax.fori_loop(0, RP * CB // LANE, yscale, 0)

        def sc_body(s, _):
            soff = s * SCH
            pltpu.sync_copy(eld_hbm.at[wid].at[pl.ds(soff, SCH)], eld)
            pltpu.sync_copy(els_hbm.at[wid].at[pl.ds(soff, SCH)], els)
            pltpu.sync_copy(elc_hbm.at[wid].at[pl.ds(soff, SCH)], elc)
            nb_s = jnp.minimum(nt - soff, SCH)
            nbat = (nb_s + GB - 1) // GB

            @pl.when(s == nsc - 1)
            def _fix():
                def fixv(v, _):
                    pos = soff + v * LANE + lanes
                    e = els[pl.ds(v * LANE, LANE)]
                    els[pl.ds(v * LANE, LANE)] = jnp.where(
                        pos < nt, e, pos & (N - 1))
                    return 0

                lax.fori_loop(0, SCH // LANE, fixv, 0)

            bufs = [(g0, sem0), (g1, sem1), (g2, sem2)]

            def mk(g, buf, sem):
                return pltpu.make_async_copy(
                    z_hbm.at[b].at[els.at[pl.ds(g * GB, GB)]], buf, sem)

            mk(0, g0, sem0).start()

            @pl.when(1 < nbat)
            def _f1():
                mk(1, g1, sem1).start()

            def batch_body(g, _):
                for par in range(3):
                    @pl.when(g % 3 == par)
                    def _p(par=par):
                        buf, sem = bufs[par]
                        mk(g, buf, sem).wait()

                        @pl.when(g + 2 < nbat)
                        def _fire():
                            nbuf, nsem = bufs[(par + 2) % 3]
                            mk(g + 2, nbuf, nsem).start()

                        process(buf, g)

                return 0

            lax.fori_loop(0, nbat, batch_body, 0)
            return 0

        lax.fori_loop(0, nsc, sc_body, 0)
        pltpu.sync_copy(acc, zo_hbm.at[b].at[pl.ds(lo * CB, RP * CB)])
        return 0

    lax.fori_loop(0, NB, block_body, 0)


def kernel(features, classification_weight):
    nby = 16
    biy = N // nby
    Y, F = pl.pallas_call(
        _yk_body,
        grid=(nby,),
        in_specs=[pl.BlockSpec((biy, D), lambda i: (i, 0)),
                  pl.BlockSpec((NCLS, D), lambda i: (0, 0))],
        out_specs=[pl.BlockSpec((biy, NCLS), lambda i: (i, 0)),
                   pl.BlockSpec((biy, D), lambda i: (i, 0))],
        out_shape=[jax.ShapeDtypeStruct((N, NCLS), jnp.float32),
                   jax.ShapeDtypeStruct((N, D), jnp.float32)],
    )(features, classification_weight)

    nbk = 32
    bik = N // nbk
    Wk, Ik = pl.pallas_call(
        _topk_body,
        grid=(nbk,),
        in_specs=[pl.BlockSpec((bik, D), lambda i: (i, 0)),
                  pl.BlockSpec((D, N), lambda i: (0, 0))],
        out_specs=[pl.BlockSpec((bik, 16), lambda i: (i, 0)),
                   pl.BlockSpec((bik, 16), lambda i: (i, 0))],
        out_shape=[jax.ShapeDtypeStruct((N, 16), jnp.float32),
                   jax.ShapeDtypeStruct((N, 16), jnp.int32)],
        scratch_shapes=[pltpu.VMEM((bik, N), jnp.float32)],
    )(F, F.T)

    idxf = Ik[:, :KNN].reshape(-1)
    wf = Wk[:, :KNN].reshape(-1)
    rowsf = jnp.repeat(jnp.arange(N, dtype=jnp.int32), KNN)

    degp = _deg_kernel(idxf, wf, rowsf)
    dinv2 = pl.pallas_call(
        _dinv_body,
        out_shape=jax.ShapeDtypeStruct((1, N), jnp.float32),
    )(degp)
    dinv = dinv2.reshape(N)

    eld, els, elc, cnts = _build_kernel(idxf, wf, rowsf, dinv)

    ypad = jnp.pad(Y, ((0, 0), (0, CPAD - NCLS)))
    y3 = ypad.reshape(N, NB, CB).transpose(1, 0, 2)
    y2 = y3.reshape(NB, N * CB)

    z3 = y3
    for _ in range(NITER):
        zo = _prop_kernel(z3, y2, eld, els, elc, cnts)
        z3 = zo.reshape(NB, N, CB)

    zfin = z3.transpose(1, 0, 2).reshape(N, CPAD)[:, :NCLS]
    return zfin


# speedup vs baseline: 4.3098x; 1.0118x over previous
import functools

import jax
import jax.numpy as jnp
from jax import lax
from jax.experimental import pallas as pl
from jax.experimental.pallas import tpu as pltpu
from jax.experimental.pallas import tpu_sc as plsc

N = 8192
D = 768
KNN = 10
ALPHA = 0.99
NU = 1.0 - ALPHA
NCLS = 1000
NITER = 10

NW = 32
RP = N // NW
NE = N * KNN
LANE = 16
CB = 256
NB = 4
CPAD = NB * CB

CS_SCAN = 4096
SCAN_BUF = 8320
CAP = 93184
SCH = 2048
GB = 64

_mesh = plsc.VectorSubcoreMesh(core_axis_name="c", subcore_axis_name="s")


def _yk_body(x_ref, w_ref, y_ref, f_ref):
    x = x_ref[...]
    nrm = jnp.sqrt(jnp.sum(x * x, axis=1, keepdims=True))
    f = x / (nrm + 1e-12)
    f_ref[...] = f
    logits = lax.dot_general(f, w_ref[...], (((1,), (1,)), ((), ())),
                             preferred_element_type=jnp.float32)
    m = jnp.max(logits, axis=1, keepdims=True)
    e = jnp.exp(logits - m)
    y_ref[...] = e / jnp.sum(e, axis=1, keepdims=True)


def _topk_body(f_ref, ft_ref, w_ref, i_ref, s_ref):
    i = pl.program_id(0)
    bi = f_ref.shape[0]
    s_ref[...] = lax.dot_general(f_ref[...], ft_ref[...],
                                 (((1,), (0,)), ((), ())),
                                 preferred_element_type=jnp.float32)
    row_g = lax.broadcasted_iota(jnp.int32, (bi, N), 0) + i * bi
    col = lax.broadcasted_iota(jnp.int32, (bi, N), 1)
    s_ref[...] = jnp.where(col == row_g, -10.0, s_ref[...])
    for k in range(KNN):
        s = s_ref[...]
        m = jnp.max(s, axis=1, keepdims=True)
        am = jnp.min(jnp.where(s == m, col, N), axis=1, keepdims=True)
        w_ref[:, k] = jnp.maximum(m[:, 0], 0.0) ** 3
        i_ref[:, k] = am[:, 0]
        s_ref[...] = jnp.where(col == am, -1e30, s)
    for k in range(KNN, 16):
        w_ref[:, k] = jnp.zeros((bi,), jnp.float32)
        i_ref[:, k] = jnp.zeros((bi,), jnp.int32)


def _dinv_body(p_ref, o_ref):
    deg = jnp.sum(p_ref[...], axis=0, keepdims=True)
    o_ref[...] = 1.0 / jnp.sqrt(deg + 1e-12)


def _wid():
    return lax.axis_index("s") * 2 + lax.axis_index("c")


@functools.partial(
    pl.kernel, mesh=_mesh,
    compiler_params=pltpu.CompilerParams(needs_layout_passes=False),
    out_type=jax.ShapeDtypeStruct((NW, N), jnp.float32),
    scratch_types=[
        pltpu.VMEM((NE // NW,), jnp.int32),
        pltpu.VMEM((NE // NW,), jnp.float32),
        pltpu.VMEM((NE // NW,), jnp.int32),
        pltpu.VMEM((N,), jnp.float32),
    ])
def _deg_kernel(idx_hbm, w_hbm, rows_hbm, degp_hbm, idxs, ws, rs, deg):
    wid = _wid()
    sl = NE // NW
    base = wid * sl
    pltpu.sync_copy(idx_hbm.at[pl.ds(base, sl)], idxs)
    pltpu.sync_copy(w_hbm.at[pl.ds(base, sl)], ws)
    pltpu.sync_copy(rows_hbm.at[pl.ds(base, sl)], rs)

    def zero_body(i, _):
        deg[pl.ds(i * LANE, LANE)] = jnp.zeros((LANE,), jnp.float32)
        return 0

    lax.fori_loop(0, N // LANE, zero_body, 0)

    def body(v, _):
        iv = idxs[pl.ds(v * LANE, LANE)]
        wv = ws[pl.ds(v * LANE, LANE)]
        rv = rs[pl.ds(v * LANE, LANE)]
        plsc.addupdate_scatter(deg, [iv], wv)
        plsc.addupdate_scatter(deg, [rv], wv)
        return 0

    lax.fori_loop(0, sl // LANE, body, 0)
    pltpu.sync_copy(deg, degp_hbm.at[wid])


@functools.partial(
    pl.kernel, mesh=_mesh,
    compiler_params=pltpu.CompilerParams(needs_layout_passes=False),
    out_type=(jax.ShapeDtypeStruct((NW, CAP), jnp.int32),
              jax.ShapeDtypeStruct((NW, CAP), jnp.int32),
              jax.ShapeDtypeStruct((NW, CAP), jnp.float32),
              jax.ShapeDtypeStruct((NW * 128,), jnp.int32)),
    scratch_types=[
        pltpu.VMEM((CS_SCAN,), jnp.int32),
        pltpu.VMEM((CS_SCAN,), jnp.float32),
        pltpu.VMEM((CS_SCAN,), jnp.int32),
        pltpu.VMEM((N,), jnp.float32),
        pltpu.VMEM((SCAN_BUF,), jnp.int32),
        pltpu.VMEM((SCAN_BUF,), jnp.int32),
        pltpu.VMEM((SCAN_BUF,), jnp.float32),
        pltpu.VMEM((LANE,), jnp.int32),
    ])
def _build_kernel(idx_hbm, w_hbm, rows_hbm, dinv_hbm, eld_hbm, els_hbm,
                  elc_hbm, cnt_hbm, idxs, ws, rws, dinv, bd, bs, bc, cvec):
    wid = _wid()
    lo = wid * RP
    pltpu.sync_copy(dinv_hbm, dinv)
    lanes = lax.iota(jnp.int32, LANE)
    nchunk = NE // CS_SCAN

    def chunk_body(ch, carry):
        cnt, off = carry
        pltpu.sync_copy(idx_hbm.at[pl.ds(ch * CS_SCAN, CS_SCAN)], idxs)
        pltpu.sync_copy(w_hbm.at[pl.ds(ch * CS_SCAN, CS_SCAN)], ws)
        pltpu.sync_copy(rows_hbm.at[pl.ds(ch * CS_SCAN, CS_SCAN)], rws)

        def vec_body(v, cnt):
            iv = idxs[pl.ds(v * LANE, LANE)]
            wv = ws[pl.ds(v * LANE, LANE)]
            rv = rws[pl.ds(v * LANE, LANE)]
            di = plsc.load_gather(dinv, [iv])
            dr = plsc.load_gather(dinv, [rv])
            coef = (ALPHA * wv) * di * dr
            m1 = (iv >= lo) & (iv < lo + RP)
            plsc.store_compressed(bd.at[pl.ds(cnt, LANE)], iv - lo, mask=m1)
            plsc.store_compressed(bs.at[pl.ds(cnt, LANE)], rv, mask=m1)
            plsc.store_compressed(bc.at[pl.ds(cnt, LANE)], coef, mask=m1)
            cnt = cnt + jnp.sum(m1.astype(jnp.int32), axis=0)
            m2 = (rv >= lo) & (rv < lo + RP)
            plsc.store_compressed(bd.at[pl.ds(cnt, LANE)], rv - lo, mask=m2)
            plsc.store_compressed(bs.at[pl.ds(cnt, LANE)], iv, mask=m2)
            plsc.store_compressed(bc.at[pl.ds(cnt, LANE)], coef, mask=m2)
            cnt = cnt + jnp.sum(m2.astype(jnp.int32), axis=0)
            return cnt

        cnt = lax.fori_loop(0, CS_SCAN // LANE, vec_body, cnt)
        offa = pl.multiple_of(off, 128)
        pltpu.sync_copy(bd, eld_hbm.at[wid].at[pl.ds(offa, SCAN_BUF)])
        pltpu.sync_copy(bs, els_hbm.at[wid].at[pl.ds(offa, SCAN_BUF)])
        pltpu.sync_copy(bc, elc_hbm.at[wid].at[pl.ds(offa, SCAN_BUF)])
        c128 = (cnt // 128) * 128
        rem = cnt - c128
        for j in range(8):
            rd = bd[pl.ds(c128 + j * LANE, LANE)]
            rs = bs[pl.ds(c128 + j * LANE, LANE)]
            rc = bc[pl.ds(c128 + j * LANE, LANE)]
            bd[pl.ds(j * LANE, LANE)] = rd
            bs[pl.ds(j * LANE, LANE)] = rs
            bc[pl.ds(j * LANE, LANE)] = rc
        return (rem, off + c128)

    cnt, off = lax.fori_loop(0, nchunk, chunk_body,
                             (jnp.int32(0), jnp.int32(0)))
    cv16 = (cnt + LANE - 1) // LANE

    def zb(i, _):
        @pl.when(i >= cv16)
        def _z():
            bd[pl.ds(i * LANE, LANE)] = jnp.zeros((LANE,), jnp.int32)
            bs[pl.ds(i * LANE, LANE)] = jnp.zeros((LANE,), jnp.int32)
            bc[pl.ds(i * LANE, LANE)] = jnp.zeros((LANE,), jnp.float32)
        return 0

    lax.fori_loop(0, SCAN_BUF // LANE, zb, 0)
    c16 = (cnt // LANE) * LANE
    tl = cnt - c16
    mtail = lanes < tl
    zd = jnp.where(mtail, bd[pl.ds(c16, LANE)], 0)
    zs = jnp.where(mtail, bs[pl.ds(c16, LANE)], 0)
    zc = jnp.where(mtail, bc[pl.ds(c16, LANE)], 0.0)
    bd[pl.ds(c16, LANE)] = zd
    bs[pl.ds(c16, LANE)] = zs
    bc[pl.ds(c16, LANE)] = zc
    offa = pl.multiple_of(off, 128)
    pltpu.sync_copy(bd, eld_hbm.at[wid].at[pl.ds(offa, SCAN_BUF)])
    pltpu.sync_copy(bs, els_hbm.at[wid].at[pl.ds(offa, SCAN_BUF)])
    pltpu.sync_copy(bc, elc_hbm.at[wid].at[pl.ds(offa, SCAN_BUF)])
    n_total = off + cnt
    cvec[pl.ds(0, LANE)] = jnp.full((LANE,), n_total, jnp.int32)
    pltpu.sync_copy(cvec, cnt_hbm.at[pl.ds(wid * 128, LANE)])


@functools.partial(
    pl.kernel, mesh=_mesh,
    compiler_params=pltpu.CompilerParams(needs_layout_passes=False),
    out_type=jax.ShapeDtypeStruct((NB, N * CB), jnp.float32),
    scratch_types=[
        pltpu.VMEM((SCH,), jnp.int32),
        pltpu.VMEM((SCH,), jnp.int32),
        pltpu.VMEM((SCH,), jnp.float32),
        pltpu.VMEM((NW * 128,), jnp.int32),
        pltpu.VMEM((GB, CB), jnp.float32),
        pltpu.VMEM((GB, CB), jnp.float32),
        pltpu.VMEM((GB, CB), jnp.float32),
        pltpu.VMEM((RP * CB,), jnp.float32),
        pltpu.SemaphoreType.DMA,
        pltpu.SemaphoreType.DMA,
        pltpu.SemaphoreType.DMA,
    ])
def _prop_kernel(z_hbm, y_hbm, eld_hbm, els_hbm, elc_hbm, cnt_hbm, zo_hbm,
                 eld, els, elc, cbuf, g0, g1, g2, acc, sem0, sem1, sem2):
    wid = _wid()
    lo = wid * RP
    lanes = lax.iota(jnp.int32, LANE)
    pltpu.sync_copy(cnt_hbm, cbuf)
    nt = jnp.max(cbuf[pl.ds(wid * 128, LANE)], axis=0)
    nsc = (nt + SCH - 1) // SCH
    ngrp = GB // LANE

    def process(gb, gbatch):
        base = gbatch * GB
        coefs = [elc[pl.ds(base + g * LANE, LANE)] for g in range(ngrp)]
        dbases = [eld[pl.ds(base + g * LANE, LANE)] * CB for g in range(ngrp)]
        rows = [lanes + g * LANE for g in range(ngrp)]

        def cbody(c, _):
            cv = (lanes + c) & (CB - 1)
            for g in range(ngrp):
                val = plsc.load_gather(gb, [rows[g], cv])
                plsc.addupdate_scatter(acc, [dbases[g] + cv], val * coefs[g])
            return 0

        lax.fori_loop(0, CB, cbody, 0, unroll=8)

    def block_body(b, _):
        pltpu.sync_copy(y_hbm.at[b].at[pl.ds(lo * CB, RP * CB)], acc)

        def yscale(i, _):
            acc[pl.ds(i * LANE, LANE)] = acc[pl.ds(i * LANE, LANE)] * NU
            return 0

        lax.fori_loop(0, RP * CB // LANE, yscale, 0)

        def sc_body(s, _):
            soff = s * SCH
            pltpu.sync_copy(eld_hbm.at[wid].at[pl.ds(soff, SCH)], eld)
            pltpu.sync_copy(els_hbm.at[wid].at[pl.ds(soff, SCH)], els)
            pltpu.sync_copy(elc_hbm.at[wid].at[pl.ds(soff, SCH)], elc)
            nb_s = jnp.minimum(nt - soff, SCH)
            nbat = (nb_s + GB - 1) // GB

            @pl.when(s == nsc - 1)
            def _fix():
                def fixv(v, _):
                    pos = soff + v * LANE + lanes
                    e = els[pl.ds(v * LANE, LANE)]
                    els[pl.ds(v * LANE, LANE)] = jnp.where(
                        pos < nt, e, pos & (N - 1))
                    return 0

                lax.fori_loop(0, SCH // LANE, fixv, 0)

            bufs = [(g0, sem0), (g1, sem1), (g2, sem2)]

            def mk(g, buf, sem):
                return pltpu.make_async_copy(
                    z_hbm.at[b].at[els.at[pl.ds(g * GB, GB)]], buf, sem)

            mk(0, g0, sem0).start()

            @pl.when(1 < nbat)
            def _f1():
                mk(1, g1, sem1).start()

            def batch_body(g, _):
                for par in range(3):
                    @pl.when(g % 3 == par)
                    def _p(par=par):
                        buf, sem = bufs[par]
                        mk(g, buf, sem).wait()

                        @pl.when(g + 2 < nbat)
                        def _fire():
                            nbuf, nsem = bufs[(par + 2) % 3]
                            mk(g + 2, nbuf, nsem).start()

                        process(buf, g)

                return 0

            lax.fori_loop(0, nbat, batch_body, 0)
            return 0

        lax.fori_loop(0, nsc, sc_body, 0)
        pltpu.sync_copy(acc, zo_hbm.at[b].at[pl.ds(lo * CB, RP * CB)])
        return 0

    lax.fori_loop(0, NB, block_body, 0)


def kernel(features, classification_weight):
    nby = 16
    biy = N // nby
    Y, F = pl.pallas_call(
        _yk_body,
        grid=(nby,),
        in_specs=[pl.BlockSpec((biy, D), lambda i: (i, 0)),
                  pl.BlockSpec((NCLS, D), lambda i: (0, 0))],
        out_specs=[pl.BlockSpec((biy, NCLS), lambda i: (i, 0)),
                   pl.BlockSpec((biy, D), lambda i: (i, 0))],
        out_shape=[jax.ShapeDtypeStruct((N, NCLS), jnp.float32),
                   jax.ShapeDtypeStruct((N, D), jnp.float32)],
    )(features, classification_weight)

    nbk = 32
    bik = N // nbk
    Wk, Ik = pl.pallas_call(
        _topk_body,
        grid=(nbk,),
        in_specs=[pl.BlockSpec((bik, D), lambda i: (i, 0)),
                  pl.BlockSpec((D, N), lambda i: (0, 0))],
        out_specs=[pl.BlockSpec((bik, 16), lambda i: (i, 0)),
                   pl.BlockSpec((bik, 16), lambda i: (i, 0))],
        out_shape=[jax.ShapeDtypeStruct((N, 16), jnp.float32),
                   jax.ShapeDtypeStruct((N, 16), jnp.int32)],
        scratch_shapes=[pltpu.VMEM((bik, N), jnp.float32)],
    )(F, F.T)

    idxf = Ik[:, :KNN].reshape(-1)
    wf = Wk[:, :KNN].reshape(-1)
    rowsf = jnp.repeat(jnp.arange(N, dtype=jnp.int32), KNN)

    degp = _deg_kernel(idxf, wf, rowsf)
    dinv2 = pl.pallas_call(
        _dinv_body,
        out_shape=jax.ShapeDtypeStruct((1, N), jnp.float32),
    )(degp)
    dinv = dinv2.reshape(N)

    eld, els, elc, cnts = _build_kernel(idxf, wf, rowsf, dinv)

    ypad = jnp.pad(Y, ((0, 0), (0, CPAD - NCLS)))
    y3 = ypad.reshape(N, NB, CB).transpose(1, 0, 2)
    y2 = y3.reshape(NB, N * CB)

    z3 = y3
    for _ in range(NITER):
        zo = _prop_kernel(z3, y2, eld, els, elc, cnts)
        z3 = zo.reshape(NB, N, CB)

    zfin = z3.transpose(1, 0, 2).reshape(N, CPAD)[:, :NCLS]
    return zfin
